# Initial kernel scaffold; baseline (speedup 1.0000x reference)
import functools

import jax
import jax.numpy as jnp
from jax import lax
from jax.experimental import pallas as pl
from jax.experimental.pallas import tpu as pltpu
from jax.experimental.pallas import tpu_sc as plsc

S, N, K, D = 16, 16384, 256, 32

NB = 2048
NBLK = N // NB
HALF_S = S // 2
CHUNK = 128
WORKERS_PER_SUB = 2
PTS_PER_WORKER = N // WORKERS_PER_SUB
NCHUNK = PTS_PER_WORKER // CHUNK
ROWS_PER_TILE = HALF_S * K // 16


def _assign_body(x_ref, c_ref, assigns_ref, ids_ref, counts_ref):
    s = pl.program_id(0)
    nb = pl.program_id(1)
    x = x_ref[0]
    c = c_ref[0]
    x2 = jnp.sum(x * x, axis=1, keepdims=True)
    c2 = jnp.sum(c * c, axis=1)
    xc = lax.dot_general(x, c, (((1,), (1,)), ((), ())),
                         preferred_element_type=jnp.float32)
    dist = x2 - 2.0 * xc + c2[None, :]
    dmin = jnp.min(dist, axis=1, keepdims=True)
    iota = lax.broadcasted_iota(jnp.int32, (NB, K), 1)
    assigns = jnp.min(jnp.where(dist == dmin, iota, K), axis=1).astype(jnp.int32)
    assigns_ref[0, 0, :] = assigns
    ids_ref[0, 0, :] = assigns + (s % HALF_S) * K
    onehot = (iota == assigns[:, None]).astype(jnp.float32)
    cblk = jnp.sum(onehot, axis=0)

    @pl.when(nb == 0)
    def _():
        counts_ref[0, 0, :] = cblk

    @pl.when(nb != 0)
    def _():
        counts_ref[0, 0, :] = counts_ref[0, 0, :] + cblk


_assign_call = pl.pallas_call(
    _assign_body,
    grid=(S, NBLK),
    in_specs=[
        pl.BlockSpec((1, NB, D), lambda s, nb: (s, nb, 0)),
        pl.BlockSpec((1, K, D), lambda s, nb: (s, 0, 0)),
    ],
    out_specs=[
        pl.BlockSpec((1, 1, NB), lambda s, nb: (s * NBLK + nb, 0, 0)),
        pl.BlockSpec((1, 1, NB), lambda s, nb: (s * NBLK + nb, 0, 0)),
        pl.BlockSpec((1, 1, K), lambda s, nb: (s, 0, 0)),
    ],
    out_shape=[
        jax.ShapeDtypeStruct((S * NBLK, 1, NB), jnp.int32),
        jax.ShapeDtypeStruct((S * NBLK, 1, NB), jnp.int32),
        jax.ShapeDtypeStruct((S, 1, K), jnp.float32),
    ],
)


_sc_mesh = plsc.VectorSubcoreMesh(core_axis_name="c", subcore_axis_name="s")


@functools.partial(
    pl.kernel,
    mesh=_sc_mesh,
    out_type=jax.ShapeDtypeStruct((S * K, D), jnp.float32),
    scratch_types=[
        pltpu.VMEM((CHUNK, D), jnp.float32),
        pltpu.VMEM((CHUNK,), jnp.int32),
        pltpu.VMEM_SHARED((HALF_S * K, D), jnp.float32),
    ],
)
def _scatter_kernel(x_hbm, ids_hbm, zeros_hbm, out_hbm, xbuf, ibuf, shared):
    cid = lax.axis_index("c")
    tid = lax.axis_index("s")
    sub = cid * HALF_S + (tid % HALF_S)
    half = tid // HALF_S
    pltpu.sync_copy(zeros_hbm.at[pl.ds(tid * ROWS_PER_TILE, ROWS_PER_TILE)],
                    shared.at[pl.ds(tid * ROWS_PER_TILE, ROWS_PER_TILE)])
    plsc.subcore_barrier()
    base_n = half * PTS_PER_WORKER

    def chunk(i, carry):
        n0 = base_n + i * CHUNK
        pltpu.sync_copy(x_hbm.at[sub, pl.ds(n0, CHUNK)], xbuf)
        pltpu.sync_copy(ids_hbm.at[sub, pl.ds(n0, CHUNK)], ibuf)
        pltpu.sync_copy(xbuf, shared.at[ibuf], add=True)
        return carry

    lax.fori_loop(0, NCHUNK, chunk, 0)
    plsc.subcore_barrier()
    out_base = cid * (HALF_S * K)
    pltpu.sync_copy(shared.at[pl.ds(tid * ROWS_PER_TILE, ROWS_PER_TILE)],
                    out_hbm.at[pl.ds(out_base + tid * ROWS_PER_TILE, ROWS_PER_TILE)])


def _mean_body(sums_ref, counts_ref, cents_ref, out_ref):
    sums = sums_ref[...]
    counts = counts_ref[...]
    means = sums / jnp.maximum(counts, 1.0)
    out_ref[...] = jnp.where(counts > 0.0, means, cents_ref[...])


_mean_call = pl.pallas_call(
    _mean_body,
    out_shape=jax.ShapeDtypeStruct((S * K, D), jnp.float32),
)


def kernel(x, centroids):
    assigns3, ids3, counts3 = _assign_call(x, centroids)
    assigns = assigns3.reshape(S, N)
    ids = ids3.reshape(S, N)
    counts = counts3.reshape(S * K, 1)
    zeros = jnp.zeros((HALF_S * K, D), jnp.float32)
    sums = _scatter_kernel(x, ids, zeros)
    new_centroids = _mean_call(sums, counts, centroids.reshape(S * K, D))
    return new_centroids.reshape(S, K, D), assigns


# speedup vs baseline: 1.5704x; 1.5704x over previous
import functools

import jax
import jax.numpy as jnp
from jax import lax
from jax.experimental import pallas as pl
from jax.experimental.pallas import tpu as pltpu
from jax.experimental.pallas import tpu_sc as plsc

S, N, K, D = 16, 16384, 256, 32

NB = 2048
NBLK = N // NB
HALF_S = S // 2
CHUNK = 128
WORKERS_PER_SUB = 2
PTS_PER_WORKER = N // WORKERS_PER_SUB
NCHUNK = PTS_PER_WORKER // CHUNK
ROWS_PER_TILE = HALF_S * K // 16


def _assign_body(x_ref, c_ref, assigns_ref, ids_ref, counts_ref):
    s = pl.program_id(0)
    nb = pl.program_id(1)
    x = x_ref[0]
    c = c_ref[0]
    x2 = jnp.sum(x * x, axis=1, keepdims=True)
    c2 = jnp.sum(c * c, axis=1)
    xc = lax.dot_general(x, c, (((1,), (1,)), ((), ())),
                         preferred_element_type=jnp.float32)
    dist = x2 - 2.0 * xc + c2[None, :]
    dmin = jnp.min(dist, axis=1, keepdims=True)
    iota = lax.broadcasted_iota(jnp.int32, (NB, K), 1)
    assigns = jnp.min(jnp.where(dist == dmin, iota, K), axis=1).astype(jnp.int32)
    assigns_ref[0, 0, :] = assigns
    ids_ref[0, 0, :] = assigns + (s % HALF_S) * K
    onehot = (iota == assigns[:, None]).astype(jnp.float32)
    cblk = jnp.sum(onehot, axis=0)

    @pl.when(nb == 0)
    def _():
        counts_ref[0, 0, :] = cblk

    @pl.when(nb != 0)
    def _():
        counts_ref[0, 0, :] = counts_ref[0, 0, :] + cblk


_assign_call = pl.pallas_call(
    _assign_body,
    grid=(S, NBLK),
    in_specs=[
        pl.BlockSpec((1, NB, D), lambda s, nb: (s, nb, 0)),
        pl.BlockSpec((1, K, D), lambda s, nb: (s, 0, 0)),
    ],
    out_specs=[
        pl.BlockSpec((1, 1, NB), lambda s, nb: (s * NBLK + nb, 0, 0)),
        pl.BlockSpec((1, 1, NB), lambda s, nb: (s * NBLK + nb, 0, 0)),
        pl.BlockSpec((1, 1, K), lambda s, nb: (s, 0, 0)),
    ],
    out_shape=[
        jax.ShapeDtypeStruct((S * NBLK, 1, NB), jnp.int32),
        jax.ShapeDtypeStruct((S * NBLK, 1, NB), jnp.int32),
        jax.ShapeDtypeStruct((S, 1, K), jnp.float32),
    ],
)


PADW = 128


def _scatter_body(x_hbm, ids_hbm, out_hbm, xbuf, xpad, ibuf, shared):
    cid = lax.axis_index("c")
    tid = lax.axis_index("s")
    sub = cid * HALF_S + (tid % HALF_S)
    half = tid // HALF_S
    zero16 = jnp.zeros((16,), jnp.float32)

    def zrow(r, carry):
        for c in range(PADW // 16):
            xpad[r, pl.ds(c * 16, 16)] = zero16
        return carry

    lax.fori_loop(0, CHUNK, zrow, 0)
    pltpu.sync_copy(xpad, shared.at[pl.ds(tid * ROWS_PER_TILE, ROWS_PER_TILE)])
    plsc.subcore_barrier()
    base_n = half * PTS_PER_WORKER

    def pad_row(r, carry):
        xpad[r, pl.ds(0, 16)] = xbuf[r, pl.ds(0, 16)]
        xpad[r, pl.ds(16, 16)] = xbuf[r, pl.ds(16, 16)]
        return carry

    def chunk(i, carry):
        n0 = base_n + i * CHUNK
        pltpu.sync_copy(x_hbm.at[sub, pl.ds(n0, CHUNK)], xbuf)
        pltpu.sync_copy(ids_hbm.at[sub, pl.ds(n0, CHUNK)], ibuf)
        lax.fori_loop(0, CHUNK, pad_row, 0)
        pltpu.sync_copy(xpad, shared.at[ibuf], add=True)
        return carry

    lax.fori_loop(0, NCHUNK, chunk, 0)
    plsc.subcore_barrier()
    out_base = cid * (HALF_S * K)
    pltpu.sync_copy(shared.at[pl.ds(tid * ROWS_PER_TILE, ROWS_PER_TILE)],
                    out_hbm.at[pl.ds(out_base + tid * ROWS_PER_TILE, ROWS_PER_TILE)])


@functools.lru_cache(maxsize=1)
def _get_scatter_kernel():
    mesh = plsc.VectorSubcoreMesh(core_axis_name="c", subcore_axis_name="s")
    return pl.kernel(
        _scatter_body,
        mesh=mesh,
        out_type=jax.ShapeDtypeStruct((S * K, PADW), jnp.float32),
        scratch_types=[
            pltpu.VMEM((CHUNK, D), jnp.float32),
            pltpu.VMEM((CHUNK, PADW), jnp.float32),
            pltpu.VMEM((CHUNK,), jnp.int32),
            pltpu.VMEM_SHARED((HALF_S * K, PADW), jnp.float32),
        ],
    )


def _mean_body(sums_ref, counts_ref, cents_ref, out_ref):
    sums = sums_ref[:, 0:D]
    counts = counts_ref[...]
    means = sums / jnp.maximum(counts, 1.0)
    out_ref[...] = jnp.where(counts > 0.0, means, cents_ref[...])


_mean_call = pl.pallas_call(
    _mean_body,
    out_shape=jax.ShapeDtypeStruct((S * K, D), jnp.float32),
)


def kernel(x, centroids):
    assigns3, ids3, counts3 = _assign_call(x, centroids)
    assigns = assigns3.reshape(S, N)
    ids = ids3.reshape(S, N)
    counts = counts3.reshape(S * K, 1)
    sums = _get_scatter_kernel()(x, ids)
    new_centroids = _mean_call(sums, counts, centroids.reshape(S * K, D))
    return new_centroids.reshape(S, K, D), assigns


# speedup vs baseline: 2.1304x; 1.3566x over previous
import functools

import jax
import jax.numpy as jnp
from jax import lax
from jax.experimental import pallas as pl
from jax.experimental.pallas import tpu as pltpu
from jax.experimental.pallas import tpu_sc as plsc

S, N, K, D = 16, 16384, 256, 32

NB = 2048
NBLK = N // NB
HALF_S = S // 2
CHUNK = 128
WORKERS_PER_SUB = 2
PTS_PER_WORKER = N // WORKERS_PER_SUB
NCHUNK = PTS_PER_WORKER // CHUNK
ROWS_PER_TILE = HALF_S * K // 16
PADW = 128


def _assign_body(x_ref, c_ref, assigns_ref):
    x = x_ref[0]
    c = c_ref[0]
    c2 = jnp.sum(c * c, axis=1)
    xc = lax.dot_general(x, c * -2.0, (((1,), (1,)), ((), ())),
                         preferred_element_type=jnp.float32)
    dist = xc + c2[None, :]
    assigns_ref[0, 0, :] = jnp.argmin(dist, axis=1).astype(jnp.int32)


_assign_call = pl.pallas_call(
    _assign_body,
    grid=(S, NBLK),
    in_specs=[
        pl.BlockSpec((1, NB, D), lambda s, nb: (s, nb, 0)),
        pl.BlockSpec((1, K, D), lambda s, nb: (s, 0, 0)),
    ],
    out_specs=pl.BlockSpec((1, 1, NB), lambda s, nb: (s * NBLK + nb, 0, 0)),
    out_shape=jax.ShapeDtypeStruct((S * NBLK, 1, NB), jnp.int32),
)


def _scatter_body(x_hbm, assigns_hbm, out_hbm, xbuf, xpad, ibuf, shared):
    cid = lax.axis_index("c")
    tid = lax.axis_index("s")
    sub = cid * HALF_S + (tid % HALF_S)
    half = tid // HALF_S
    row_off = (tid % HALF_S) * K
    zero16 = jnp.zeros((16,), jnp.float32)
    lane16 = lax.broadcasted_iota(jnp.int32, (16,), 0)
    one_hot16 = jnp.where(lane16 == 0, 1.0, 0.0).astype(jnp.float32)

    def zrow(r, carry):
        for cc in range(PADW // 16):
            xpad[r, pl.ds(cc * 16, 16)] = zero16
        return carry

    lax.fori_loop(0, CHUNK, zrow, 0)
    pltpu.sync_copy(xpad, shared.at[pl.ds(tid * ROWS_PER_TILE, ROWS_PER_TILE)])

    def onerow(r, carry):
        xpad[r, pl.ds(D, 16)] = one_hot16
        return carry

    lax.fori_loop(0, CHUNK, onerow, 0)
    plsc.subcore_barrier()
    base_n = half * PTS_PER_WORKER

    def pad_row(r, carry):
        xpad[r, pl.ds(0, 16)] = xbuf[r, pl.ds(0, 16)]
        xpad[r, pl.ds(16, 16)] = xbuf[r, pl.ds(16, 16)]
        return carry

    def chunk(i, carry):
        n0 = base_n + i * CHUNK
        pltpu.sync_copy(x_hbm.at[sub, pl.ds(n0, CHUNK)], xbuf)
        pltpu.sync_copy(assigns_hbm.at[sub, pl.ds(n0, CHUNK)], ibuf)
        lax.fori_loop(0, CHUNK, pad_row, 0)
        for j in range(CHUNK // 16):
            sl = pl.ds(j * 16, 16)
            ibuf[sl] = ibuf[sl] + row_off
        pltpu.sync_copy(xpad, shared.at[ibuf], add=True)
        return carry

    lax.fori_loop(0, NCHUNK, chunk, 0)
    plsc.subcore_barrier()
    out_base = cid * (HALF_S * K)
    pltpu.sync_copy(shared.at[pl.ds(tid * ROWS_PER_TILE, ROWS_PER_TILE)],
                    out_hbm.at[pl.ds(out_base + tid * ROWS_PER_TILE, ROWS_PER_TILE)])


@functools.lru_cache(maxsize=1)
def _get_scatter_kernel():
    mesh = plsc.VectorSubcoreMesh(core_axis_name="c", subcore_axis_name="s")
    return pl.kernel(
        _scatter_body,
        mesh=mesh,
        out_type=jax.ShapeDtypeStruct((S * K, PADW), jnp.float32),
        scratch_types=[
            pltpu.VMEM((CHUNK, D), jnp.float32),
            pltpu.VMEM((CHUNK, PADW), jnp.float32),
            pltpu.VMEM((CHUNK,), jnp.int32),
            pltpu.VMEM_SHARED((HALF_S * K, PADW), jnp.float32),
        ],
    )


def _mean_body(sums_ref, cents_ref, out_ref):
    sums = sums_ref[:, 0:D]
    counts = sums_ref[:, D:D + 1]
    means = sums / jnp.maximum(counts, 1.0)
    out_ref[...] = jnp.where(counts > 0.0, means, cents_ref[...])


_mean_call = pl.pallas_call(
    _mean_body,
    out_shape=jax.ShapeDtypeStruct((S * K, D), jnp.float32),
)


def kernel(x, centroids):
    assigns3 = _assign_call(x, centroids)
    assigns = assigns3.reshape(S, N)
    sums = _get_scatter_kernel()(x, assigns)
    new_centroids = _mean_call(sums, centroids.reshape(S * K, D))
    return new_centroids.reshape(S, K, D), assigns


# speedup vs baseline: 2.5758x; 1.2091x over previous
import functools

import jax
import jax.numpy as jnp
from jax import lax
from jax.experimental import pallas as pl
from jax.experimental.pallas import tpu as pltpu
from jax.experimental.pallas import tpu_sc as plsc

S, N, K, D = 16, 16384, 256, 32

NB = 4096
NBLK = N // NB
HALF_S = S // 2
CHUNK = 128
WORKERS_PER_SUB = 2
PTS_PER_WORKER = N // WORKERS_PER_SUB
NCHUNK = PTS_PER_WORKER // CHUNK
ROWS_PER_TILE = HALF_S * K // 16
PADW = 128


def _assign_body(x_ref, c_ref, assigns_ref):
    x = x_ref[0]
    c = c_ref[0]
    c2 = jnp.sum(c * c, axis=1)
    xc = lax.dot_general(x, c * -2.0, (((1,), (1,)), ((), ())),
                         preferred_element_type=jnp.float32)
    dist = xc + c2[None, :]
    assigns_ref[0, 0, :] = jnp.argmin(dist, axis=1).astype(jnp.int32)


_assign_call = pl.pallas_call(
    _assign_body,
    grid=(S, NBLK),
    in_specs=[
        pl.BlockSpec((1, NB, D), lambda s, nb: (s, nb, 0)),
        pl.BlockSpec((1, K, D), lambda s, nb: (s, 0, 0)),
    ],
    out_specs=pl.BlockSpec((1, 1, NB), lambda s, nb: (s * NBLK + nb, 0, 0)),
    out_shape=jax.ShapeDtypeStruct((S * NBLK, 1, NB), jnp.int32),
)


def _scatter_body(x_hbm, assigns_hbm, out_hbm,
                  xbuf0, xbuf1, xpad0, xpad1, ibuf0, ibuf1,
                  sem_x0, sem_x1, sem_i0, sem_i1, sem_s0, sem_s1, shared):
    cid = lax.axis_index("c")
    tid = lax.axis_index("s")
    sub = cid * HALF_S + (tid % HALF_S)
    half = tid // HALF_S
    row_off = (tid % HALF_S) * K
    zero16 = jnp.zeros((16,), jnp.float32)
    lane16 = lax.broadcasted_iota(jnp.int32, (16,), 0)
    one_hot16 = jnp.where(lane16 == 0, 1.0, 0.0).astype(jnp.float32)
    xbufs, xpads, ibufs = (xbuf0, xbuf1), (xpad0, xpad1), (ibuf0, ibuf1)
    sems_x, sems_i, sems_s = (sem_x0, sem_x1), (sem_i0, sem_i1), (sem_s0, sem_s1)

    def zrow(r, carry):
        for cc in range(PADW // 16):
            xpad0[r, pl.ds(cc * 16, 16)] = zero16
        return carry

    lax.fori_loop(0, CHUNK, zrow, 0)
    pltpu.sync_copy(xpad0, shared.at[pl.ds(tid * ROWS_PER_TILE, ROWS_PER_TILE)])

    def onerow(r, carry):
        xpad0[r, pl.ds(D, 16)] = one_hot16
        xpad1[r, pl.ds(D, 16)] = one_hot16
        for cc in range(D // 16 + 1, PADW // 16):
            xpad1[r, pl.ds(cc * 16, 16)] = zero16
        return carry

    lax.fori_loop(0, CHUNK, onerow, 0)
    plsc.subcore_barrier()
    base_n = half * PTS_PER_WORKER

    def start_fetch(i, b):
        n0 = base_n + i * CHUNK
        pltpu.make_async_copy(x_hbm.at[sub, pl.ds(n0, CHUNK)], xbufs[b],
                              sems_x[b]).start()
        pltpu.make_async_copy(assigns_hbm.at[sub, pl.ds(n0, CHUNK)], ibufs[b],
                              sems_i[b]).start()

    def wait_fetch(i, b):
        n0 = base_n + i * CHUNK
        pltpu.make_async_copy(x_hbm.at[sub, pl.ds(n0, CHUNK)], xbufs[b],
                              sems_x[b]).wait()
        pltpu.make_async_copy(assigns_hbm.at[sub, pl.ds(n0, CHUNK)], ibufs[b],
                              sems_i[b]).wait()

    def do_chunk(i, b, drain_other):
        wait_fetch(i, b)

        def pad_row(r, carry):
            xpads[b][r, pl.ds(0, 16)] = xbufs[b][r, pl.ds(0, 16)]
            xpads[b][r, pl.ds(16, 16)] = xbufs[b][r, pl.ds(16, 16)]
            return carry

        lax.fori_loop(0, CHUNK, pad_row, 0)
        for j in range(CHUNK // 16):
            sl = pl.ds(j * 16, 16)
            ibufs[b][sl] = ibufs[b][sl] + row_off
        pltpu.make_async_copy(xpads[b], shared.at[ibufs[b]], sems_s[b]).start(add=True)
        @pl.when(i + 1 < NCHUNK)
        def _():
            if drain_other:
                pltpu.make_async_copy(
                    xpads[1 - b], shared.at[ibufs[1 - b]], sems_s[1 - b]).wait()
            start_fetch(i + 1, 1 - b)

    start_fetch(0, 0)
    do_chunk(0, 0, False)
    do_chunk(1, 1, True)

    def pair(g, carry):
        i = 2 + g * 2
        do_chunk(i, 0, True)
        do_chunk(i + 1, 1, True)
        return carry

    lax.fori_loop(0, (NCHUNK - 2) // 2, pair, 0)
    pltpu.make_async_copy(xpads[0], shared.at[ibufs[0]], sems_s[0]).wait()
    pltpu.make_async_copy(xpads[1], shared.at[ibufs[1]], sems_s[1]).wait()
    plsc.subcore_barrier()
    out_base = cid * (HALF_S * K)
    pltpu.sync_copy(shared.at[pl.ds(tid * ROWS_PER_TILE, ROWS_PER_TILE)],
                    out_hbm.at[pl.ds(out_base + tid * ROWS_PER_TILE, ROWS_PER_TILE)])


@functools.lru_cache(maxsize=1)
def _get_scatter_kernel():
    mesh = plsc.VectorSubcoreMesh(core_axis_name="c", subcore_axis_name="s")
    return pl.kernel(
        _scatter_body,
        mesh=mesh,
        out_type=jax.ShapeDtypeStruct((S * K, PADW), jnp.float32),
        scratch_types=[
            pltpu.VMEM((CHUNK, D), jnp.float32),
            pltpu.VMEM((CHUNK, D), jnp.float32),
            pltpu.VMEM((CHUNK, PADW), jnp.float32),
            pltpu.VMEM((CHUNK, PADW), jnp.float32),
            pltpu.VMEM((CHUNK,), jnp.int32),
            pltpu.VMEM((CHUNK,), jnp.int32),
            pltpu.SemaphoreType.DMA,
            pltpu.SemaphoreType.DMA,
            pltpu.SemaphoreType.DMA,
            pltpu.SemaphoreType.DMA,
            pltpu.SemaphoreType.DMA,
            pltpu.SemaphoreType.DMA,
            pltpu.VMEM_SHARED((HALF_S * K, PADW), jnp.float32),
        ],
    )


def _mean_body(sums_ref, cents_ref, out_ref):
    sums = sums_ref[:, 0:D]
    counts = sums_ref[:, D:D + 1]
    means = sums / jnp.maximum(counts, 1.0)
    out_ref[...] = jnp.where(counts > 0.0, means, cents_ref[...])


_mean_call = pl.pallas_call(
    _mean_body,
    out_shape=jax.ShapeDtypeStruct((S * K, D), jnp.float32),
)


def kernel(x, centroids):
    assigns3 = _assign_call(x, centroids)
    assigns = assigns3.reshape(S, N)
    sums = _get_scatter_kernel()(x, assigns)
    new_centroids = _mean_call(sums, centroids.reshape(S * K, D))
    return new_centroids.reshape(S, K, D), assigns


# speedup vs baseline: 2.6037x; 1.0109x over previous
import functools

import jax
import jax.numpy as jnp
from jax import lax
from jax.experimental import pallas as pl
from jax.experimental.pallas import tpu as pltpu
from jax.experimental.pallas import tpu_sc as plsc

S, N, K, D = 16, 16384, 256, 32

NB = 4096
NBLK = N // NB
HALF_S = S // 2
CHUNK = 128
WORKERS_PER_SUB = 2
PTS_PER_WORKER = N // WORKERS_PER_SUB
NCHUNK = PTS_PER_WORKER // CHUNK
ROWS_PER_TILE = HALF_S * K // 16
PADW = 128


def _assign_body(x_ref, c_ref, assigns_ref):
    x = x_ref[0]
    c = c_ref[0]
    c2 = jnp.sum(c * c, axis=1)
    xc = lax.dot_general(x, c * -2.0, (((1,), (1,)), ((), ())),
                         preferred_element_type=jnp.float32)
    dist = xc + c2[None, :]
    assigns_ref[0, 0, :] = jnp.argmin(dist, axis=1).astype(jnp.int32)


_assign_call = pl.pallas_call(
    _assign_body,
    grid=(S, NBLK),
    in_specs=[
        pl.BlockSpec((1, NB, D), lambda s, nb: (s, nb, 0)),
        pl.BlockSpec((1, K, D), lambda s, nb: (s, 0, 0)),
    ],
    out_specs=pl.BlockSpec((1, 1, NB), lambda s, nb: (s * NBLK + nb, 0, 0)),
    out_shape=jax.ShapeDtypeStruct((S * NBLK, 1, NB), jnp.int32),
)


def _scatter_body(x_hbm, assigns_hbm, cents_hbm, out_hbm,
                  xbuf0, xbuf1, xpad0, xpad1, ibuf0, ibuf1,
                  sem_x0, sem_x1, sem_i0, sem_i1, sem_s0, sem_s1, shared):
    cid = lax.axis_index("c")
    tid = lax.axis_index("s")
    sub = cid * HALF_S + (tid % HALF_S)
    half = tid // HALF_S
    row_off = (tid % HALF_S) * K
    zero16 = jnp.zeros((16,), jnp.float32)
    one16 = jnp.zeros((16,), jnp.float32) + 1.0
    xbufs, xpads, ibufs = (xbuf0, xbuf1), (xpad0, xpad1), (ibuf0, ibuf1)
    sems_x, sems_i, sems_s = (sem_x0, sem_x1), (sem_i0, sem_i1), (sem_s0, sem_s1)

    def zrow(r, carry):
        for cc in range(PADW // 16):
            xpad0[r, pl.ds(cc * 16, 16)] = zero16
        return carry

    lax.fori_loop(0, CHUNK, zrow, 0)
    pltpu.sync_copy(xpad0, shared.at[pl.ds(tid * ROWS_PER_TILE, ROWS_PER_TILE)])

    def onerow(r, carry):
        xpad0[r, pl.ds(D, 16)] = one16
        xpad1[r, pl.ds(D, 16)] = one16
        for cc in range(D // 16 + 1, PADW // 16):
            xpad1[r, pl.ds(cc * 16, 16)] = zero16
        return carry

    lax.fori_loop(0, CHUNK, onerow, 0)
    plsc.subcore_barrier()
    base_n = half * PTS_PER_WORKER

    def start_fetch(i, b):
        n0 = base_n + i * CHUNK
        pltpu.make_async_copy(x_hbm.at[sub, pl.ds(n0, CHUNK)], xbufs[b],
                              sems_x[b]).start()
        pltpu.make_async_copy(assigns_hbm.at[sub, pl.ds(n0, CHUNK)], ibufs[b],
                              sems_i[b]).start()

    def wait_fetch(i, b):
        n0 = base_n + i * CHUNK
        pltpu.make_async_copy(x_hbm.at[sub, pl.ds(n0, CHUNK)], xbufs[b],
                              sems_x[b]).wait()
        pltpu.make_async_copy(assigns_hbm.at[sub, pl.ds(n0, CHUNK)], ibufs[b],
                              sems_i[b]).wait()

    def do_chunk(i, b, drain_other):
        wait_fetch(i, b)

        def pad_row(r, carry):
            xpads[b][r, pl.ds(0, 16)] = xbufs[b][r, pl.ds(0, 16)]
            xpads[b][r, pl.ds(16, 16)] = xbufs[b][r, pl.ds(16, 16)]
            return carry

        lax.fori_loop(0, CHUNK, pad_row, 0)
        for j in range(CHUNK // 16):
            sl = pl.ds(j * 16, 16)
            ibufs[b][sl] = ibufs[b][sl] + row_off
        pltpu.make_async_copy(xpads[b], shared.at[ibufs[b]], sems_s[b]).start(add=True)
        @pl.when(i + 1 < NCHUNK)
        def _():
            if drain_other:
                pltpu.make_async_copy(
                    xpads[1 - b], shared.at[ibufs[1 - b]], sems_s[1 - b]).wait()
            start_fetch(i + 1, 1 - b)

    start_fetch(0, 0)
    do_chunk(0, 0, False)
    do_chunk(1, 1, True)

    def pair(g, carry):
        i = 2 + g * 2
        do_chunk(i, 0, True)
        do_chunk(i + 1, 1, True)
        return carry

    lax.fori_loop(0, (NCHUNK - 2) // 2, pair, 0)
    pltpu.make_async_copy(xpads[0], shared.at[ibufs[0]], sems_s[0]).wait()
    pltpu.make_async_copy(xpads[1], shared.at[ibufs[1]], sems_s[1]).wait()
    plsc.subcore_barrier()
    out_base = cid * (HALF_S * K)
    rows0 = out_base + tid * ROWS_PER_TILE
    pltpu.sync_copy(shared.at[pl.ds(tid * ROWS_PER_TILE, ROWS_PER_TILE)], xpad0)
    pltpu.sync_copy(cents_hbm.at[pl.ds(rows0, ROWS_PER_TILE)], xbuf0)

    def mrow(r, carry):
        cntv = xpad0[r, pl.ds(D, 16)]
        denom = jnp.maximum(cntv, one16)
        keep = jnp.minimum(cntv, one16)
        drop = one16 - keep
        m0 = xpad0[r, pl.ds(0, 16)] / denom
        m1 = xpad0[r, pl.ds(16, 16)] / denom
        xbuf0[r, pl.ds(0, 16)] = m0 * keep + xbuf0[r, pl.ds(0, 16)] * drop
        xbuf0[r, pl.ds(16, 16)] = m1 * keep + xbuf0[r, pl.ds(16, 16)] * drop
        return carry

    lax.fori_loop(0, ROWS_PER_TILE, mrow, 0)
    pltpu.sync_copy(xbuf0, out_hbm.at[pl.ds(rows0, ROWS_PER_TILE)])


@functools.lru_cache(maxsize=1)
def _get_scatter_kernel():
    mesh = plsc.VectorSubcoreMesh(core_axis_name="c", subcore_axis_name="s")
    return pl.kernel(
        _scatter_body,
        mesh=mesh,
        out_type=jax.ShapeDtypeStruct((S * K, D), jnp.float32),
        scratch_types=[
            pltpu.VMEM((CHUNK, D), jnp.float32),
            pltpu.VMEM((CHUNK, D), jnp.float32),
            pltpu.VMEM((CHUNK, PADW), jnp.float32),
            pltpu.VMEM((CHUNK, PADW), jnp.float32),
            pltpu.VMEM((CHUNK,), jnp.int32),
            pltpu.VMEM((CHUNK,), jnp.int32),
            pltpu.SemaphoreType.DMA,
            pltpu.SemaphoreType.DMA,
            pltpu.SemaphoreType.DMA,
            pltpu.SemaphoreType.DMA,
            pltpu.SemaphoreType.DMA,
            pltpu.SemaphoreType.DMA,
            pltpu.VMEM_SHARED((HALF_S * K, PADW), jnp.float32),
        ],
    )


def kernel(x, centroids):
    assigns3 = _assign_call(x, centroids)
    assigns = assigns3.reshape(S, N)
    new_centroids = _get_scatter_kernel()(x, assigns, centroids.reshape(S * K, D))
    return new_centroids.reshape(S, K, D), assigns
